# baseline (speedup 1.0000x reference)
import functools

import jax
import jax.numpy as jnp
from jax import lax
from jax.experimental import pallas as pl
from jax.experimental.pallas import tpu as pltpu
from jax.experimental.pallas import tpu_sc as plsc

_FEAT = 768
_HID = 256
_K = 8192
_NS = 10
_MOM = 0.999
_N = 4608

_BT_A = 512
_BT_B = 128
_BK = 1024

_NW = 32
_TW = _N // _NW
_C = 24
_NCH = _TW // _C


def _prep_body(cent_ref, cn_ref, c2_ref):
    c = cent_ref[...]
    nrm = jnp.sqrt(jnp.sum(c * c, axis=1, keepdims=True))
    cn = c / (nrm + 1e-12)
    cn_ref[...] = cn
    c2_ref[...] = jnp.sum(cn * cn, axis=1, keepdims=True)


def _head_body(xo_ref, xa_ref, w1_ref, b1_ref, w2_ref, b2_ref,
               ew1_ref, eb1_ref, ew2_ref, eb2_ref,
               z1_ref, n1_ref, z2_ref, loss_ref):
    i = pl.program_id(0)
    w1 = w1_ref[...]
    b1 = b1_ref[...]
    w2 = w2_ref[...]
    b2 = b2_ref[...]
    xo = xo_ref[...]
    h1 = jnp.maximum(jnp.dot(xo, w1, preferred_element_type=jnp.float32) + b1, 0.0)
    z1 = jnp.dot(h1, w2, preferred_element_type=jnp.float32) + b2
    z1_ref[...] = z1
    nr1 = jnp.sqrt(jnp.sum(z1 * z1, axis=1, keepdims=True))
    n1 = z1 / (nr1 + 1e-12)
    n1_ref[...] = n1
    z2_ref[...] = jnp.sum(n1 * n1, axis=1, keepdims=True)
    uw1 = ew1_ref[...] * _MOM + w1 * (1.0 - _MOM)
    ub1 = eb1_ref[...] * _MOM + b1 * (1.0 - _MOM)
    uw2 = ew2_ref[...] * _MOM + w2 * (1.0 - _MOM)
    ub2 = eb2_ref[...] * _MOM + b2 * (1.0 - _MOM)
    xa = xa_ref[...]
    h2 = jnp.maximum(jnp.dot(xa, uw1, preferred_element_type=jnp.float32) + ub1, 0.0)
    za = jnp.dot(h2, uw2, preferred_element_type=jnp.float32) + ub2
    nr2 = jnp.sqrt(jnp.sum(za * za, axis=1, keepdims=True))
    n2 = za / (nr2 + 1e-12)
    d = n1 - n2
    part = jnp.sum(d * d).reshape(1, 1)
    @pl.when(i == 0)
    def _():
        loss_ref[...] = jnp.zeros((1, 1), jnp.float32)
    loss_ref[...] += part
    @pl.when(i == pl.num_programs(0) - 1)
    def _():
        loss_ref[...] = loss_ref[...] / float(_N * _HID)


def _vq_body(n1_ref, z2_ref, cn_ref, c2_ref, idx_ref, idx2_ref):
    mm = lax.dot_general(n1_ref[...], cn_ref[...], (((1,), (1,)), ((), ())),
                         preferred_element_type=jnp.float32)
    dist = (z2_ref[...] + c2_ref[...]) - 2.0 * mm
    dmin = jnp.min(dist, axis=1, keepdims=True)
    ii = lax.broadcasted_iota(jnp.int32, dist.shape, 1)
    cand = jnp.where(dist == dmin, ii, _K)
    amin = jnp.min(cand, axis=1, keepdims=True)
    idx_ref[...] = amin
    idx2_ref[...] = amin * _NS + lax.broadcasted_iota(
        jnp.int32, (amin.shape[0], _NS), 1)


_CH = _C // 2
_R = _CH * _NS


@functools.cache
def _make_sc_gather():
    @functools.partial(
        pl.kernel,
        out_type=[jax.ShapeDtypeStruct((_N * _NS, _HID), jnp.float32),
                  jax.ShapeDtypeStruct((_N, _HID), jnp.float32)],
        mesh=plsc.VectorSubcoreMesh(core_axis_name="c", subcore_axis_name="s"),
        scratch_types=[
            pltpu.VMEM((_C,), jnp.int32),
            pltpu.VMEM((_R,), jnp.int32),
            pltpu.VMEM((_R,), jnp.int32),
            pltpu.VMEM((_R, _HID), jnp.float32),
            pltpu.VMEM((_R, _HID), jnp.float32),
            pltpu.VMEM((_C, _HID), jnp.float32),
            pltpu.SemaphoreType.DMA,
            pltpu.SemaphoreType.DMA,
            pltpu.SemaphoreType.DMA,
        ],
    )
    def _sc_gather(queue2_hbm, cent_hbm, idx_hbm, idx2_hbm, z1_hbm,
                   pos_hbm, proxy_hbm,
                   ic_v, ia_v, ib_v, qbufa, qbufb, cbuf, semq, semc, semz):
        wid = lax.axis_index("s") * 2 + lax.axis_index("c")
        base = wid * _TW

        def chunk(j, _):
            tb = base + j * _C
            fb = tb * _NS
            pltpu.sync_copy(idx_hbm.at[pl.ds(tb, _C)], ic_v)
            pltpu.sync_copy(idx2_hbm.at[pl.ds(fb, _R)], ia_v)
            pltpu.sync_copy(idx2_hbm.at[pl.ds(fb + _R, _R)], ib_v)
            ca = pltpu.async_copy(queue2_hbm.at[ia_v], qbufa, semq)
            cb = pltpu.async_copy(queue2_hbm.at[ib_v], qbufb, semq)
            cc = pltpu.async_copy(cent_hbm.at[ic_v], cbuf, semc)
            ca.wait()
            cb.wait()
            zw = []
            for t in range(_CH):
                zw.append(pltpu.async_copy(
                    z1_hbm.at[pl.ds(tb + t, 1)],
                    qbufa.at[pl.ds(t * _NS + _NS - 1, 1)], semz))
                zw.append(pltpu.async_copy(
                    z1_hbm.at[pl.ds(tb + _CH + t, 1)],
                    qbufb.at[pl.ds(t * _NS + _NS - 1, 1)], semz))
            for w in zw:
                w.wait()
            pltpu.sync_copy(qbufa, pos_hbm.at[pl.ds(fb, _R)])
            pltpu.sync_copy(qbufb, pos_hbm.at[pl.ds(fb + _R, _R)])
            cc.wait()
            pltpu.sync_copy(cbuf, proxy_hbm.at[pl.ds(tb, _C)])
            return ()

        lax.fori_loop(0, _NCH, chunk, ())

    return _sc_gather


def kernel(img, aug_img, W1, b1, W2, b2, eW1, eb1, eW2, eb2, centroid, queue):
    xo = jnp.transpose(img, (0, 2, 3, 1)).reshape(-1, _FEAT)
    xa = jnp.transpose(aug_img, (0, 2, 3, 1)).reshape(-1, _FEAT)
    b1r = b1.reshape(1, _FEAT)
    b2r = b2.reshape(1, _HID)
    eb1r = eb1.reshape(1, _FEAT)
    eb2r = eb2.reshape(1, _HID)

    cn, c2 = pl.pallas_call(
        _prep_body,
        grid=(_K // _BK,),
        in_specs=[pl.BlockSpec((_BK, _HID), lambda i: (i, 0))],
        out_specs=[pl.BlockSpec((_BK, _HID), lambda i: (i, 0)),
                   pl.BlockSpec((_BK, 1), lambda i: (i, 0))],
        out_shape=[jax.ShapeDtypeStruct((_K, _HID), jnp.float32),
                   jax.ShapeDtypeStruct((_K, 1), jnp.float32)],
    )(centroid)

    z1, n1, z2, loss = pl.pallas_call(
        _head_body,
        grid=(_N // _BT_A,),
        in_specs=[pl.BlockSpec((_BT_A, _FEAT), lambda i: (i, 0)),
                  pl.BlockSpec((_BT_A, _FEAT), lambda i: (i, 0)),
                  pl.BlockSpec((_FEAT, _FEAT), lambda i: (0, 0)),
                  pl.BlockSpec((1, _FEAT), lambda i: (0, 0)),
                  pl.BlockSpec((_FEAT, _HID), lambda i: (0, 0)),
                  pl.BlockSpec((1, _HID), lambda i: (0, 0)),
                  pl.BlockSpec((_FEAT, _FEAT), lambda i: (0, 0)),
                  pl.BlockSpec((1, _FEAT), lambda i: (0, 0)),
                  pl.BlockSpec((_FEAT, _HID), lambda i: (0, 0)),
                  pl.BlockSpec((1, _HID), lambda i: (0, 0))],
        out_specs=[pl.BlockSpec((_BT_A, _HID), lambda i: (i, 0)),
                   pl.BlockSpec((_BT_A, _HID), lambda i: (i, 0)),
                   pl.BlockSpec((_BT_A, 1), lambda i: (i, 0)),
                   pl.BlockSpec((1, 1), lambda i: (0, 0))],
        out_shape=[jax.ShapeDtypeStruct((_N, _HID), jnp.float32),
                   jax.ShapeDtypeStruct((_N, _HID), jnp.float32),
                   jax.ShapeDtypeStruct((_N, 1), jnp.float32),
                   jax.ShapeDtypeStruct((1, 1), jnp.float32)],
    )(xo, xa, W1, b1r, W2, b2r, eW1, eb1r, eW2, eb2r)

    idx, idx2 = pl.pallas_call(
        _vq_body,
        grid=(_N // _BT_B,),
        in_specs=[pl.BlockSpec((_BT_B, _HID), lambda i: (i, 0)),
                  pl.BlockSpec((_BT_B, 1), lambda i: (i, 0)),
                  pl.BlockSpec((_K, _HID), lambda i: (0, 0)),
                  pl.BlockSpec((1, _K), lambda i: (0, 0))],
        out_specs=[pl.BlockSpec((_BT_B, 1), lambda i: (i, 0)),
                   pl.BlockSpec((_BT_B, _NS), lambda i: (i, 0))],
        out_shape=[jax.ShapeDtypeStruct((_N, 1), jnp.int32),
                   jax.ShapeDtypeStruct((_N, _NS), jnp.int32)],
    )(n1, z2, cn, c2.reshape(1, _K))

    pos_flat, pos_proxy = _make_sc_gather()(
        queue.reshape(_K * _NS, _HID), centroid, idx.reshape(_N),
        idx2.reshape(_N * _NS), z1)
    positives = pos_flat.reshape(_N, _NS, _HID)

    out = jnp.transpose(n1.reshape(8, 24, 24, _HID), (0, 3, 1, 2))
    loss1 = loss[0, 0]
    return (out, pos_proxy, positives, loss1)


# speedup vs baseline: 1.4041x; 1.4041x over previous
import functools

import jax
import jax.numpy as jnp
from jax import lax
from jax.experimental import pallas as pl
from jax.experimental.pallas import tpu as pltpu
from jax.experimental.pallas import tpu_sc as plsc

_FEAT = 768
_HID = 256
_K = 8192
_NS = 10
_MOM = 0.999
_N = 4608

_BT_A = 512
_BT_B = 128
_BK = 1024

_NW = 32
_TW = _N // _NW
_C = 24
_NCH = _TW // _C


def _prep_body(cent_ref, cn_ref, c2_ref):
    c = cent_ref[...]
    nrm = jnp.sqrt(jnp.sum(c * c, axis=1, keepdims=True))
    cn = c / (nrm + 1e-12)
    cn_ref[...] = cn
    c2_ref[...] = jnp.sum(cn * cn, axis=1, keepdims=True)


def _head_body(xo_ref, xa_ref, w1_ref, b1_ref, w2_ref, b2_ref,
               ew1_ref, eb1_ref, ew2_ref, eb2_ref,
               z1_ref, n1_ref, z2_ref, loss_ref):
    i = pl.program_id(0)
    w1 = w1_ref[...]
    b1 = b1_ref[...]
    w2 = w2_ref[...]
    b2 = b2_ref[...]
    xo = xo_ref[...]
    h1 = jnp.maximum(jnp.dot(xo, w1, preferred_element_type=jnp.float32) + b1, 0.0)
    z1 = jnp.dot(h1, w2, preferred_element_type=jnp.float32) + b2
    z1_ref[...] = z1
    nr1 = jnp.sqrt(jnp.sum(z1 * z1, axis=1, keepdims=True))
    n1 = z1 / (nr1 + 1e-12)
    n1_ref[...] = n1
    z2_ref[...] = jnp.sum(n1 * n1, axis=1, keepdims=True)
    uw1 = ew1_ref[...] * _MOM + w1 * (1.0 - _MOM)
    ub1 = eb1_ref[...] * _MOM + b1 * (1.0 - _MOM)
    uw2 = ew2_ref[...] * _MOM + w2 * (1.0 - _MOM)
    ub2 = eb2_ref[...] * _MOM + b2 * (1.0 - _MOM)
    xa = xa_ref[...]
    h2 = jnp.maximum(jnp.dot(xa, uw1, preferred_element_type=jnp.float32) + ub1, 0.0)
    za = jnp.dot(h2, uw2, preferred_element_type=jnp.float32) + ub2
    nr2 = jnp.sqrt(jnp.sum(za * za, axis=1, keepdims=True))
    n2 = za / (nr2 + 1e-12)
    d = n1 - n2
    part = jnp.sum(d * d).reshape(1, 1)
    @pl.when(i == 0)
    def _():
        loss_ref[...] = jnp.zeros((1, 1), jnp.float32)
    loss_ref[...] += part
    @pl.when(i == pl.num_programs(0) - 1)
    def _():
        loss_ref[...] = loss_ref[...] / float(_N * _HID)


def _vq_body(n1_ref, z2_ref, cn_ref, c2_ref, idx_ref):
    mm = lax.dot_general(n1_ref[...], cn_ref[...], (((1,), (1,)), ((), ())),
                         preferred_element_type=jnp.float32)
    dist = (z2_ref[...] + c2_ref[...]) - 2.0 * mm
    dmin = jnp.min(dist, axis=1, keepdims=True)
    ii = lax.broadcasted_iota(jnp.int32, dist.shape, 1)
    cand = jnp.where(dist == dmin, ii, _K)
    idx_ref[...] = jnp.min(cand, axis=1, keepdims=True)


@functools.cache
def _make_sc_gather():
    @functools.partial(
        pl.kernel,
        out_type=[jax.ShapeDtypeStruct((_N, _NS, _HID), jnp.float32),
                  jax.ShapeDtypeStruct((_N, _HID), jnp.float32)],
        mesh=plsc.VectorSubcoreMesh(core_axis_name="c", subcore_axis_name="s"),
        scratch_types=[
            pltpu.VMEM((_C,), jnp.int32),
            pltpu.VMEM((_NS, _C, _HID), jnp.float32),
            pltpu.VMEM((_C, _HID), jnp.float32),
            pltpu.SemaphoreType.DMA,
            pltpu.SemaphoreType.DMA,
        ],
    )
    def _sc_gather(queue_hbm, cent_hbm, idx_hbm, z1_hbm, pos_hbm, proxy_hbm,
                   ic_v, qbuf, cbuf, semq, semc):
        wid = lax.axis_index("s") * 2 + lax.axis_index("c")
        base = wid * _TW

        def chunk(j, _):
            tb = base + j * _C
            pltpu.sync_copy(idx_hbm.at[pl.ds(tb, _C)], ic_v)
            cps = [pltpu.async_copy(queue_hbm.at[:, s].at[ic_v], qbuf.at[s],
                                    semq)
                   for s in range(_NS - 1)]
            cps.append(pltpu.async_copy(
                z1_hbm.at[pl.ds(tb, _C)], qbuf.at[_NS - 1], semq))
            cc = pltpu.async_copy(cent_hbm.at[ic_v], cbuf, semc)
            for w in cps:
                w.wait()
            for s in range(_NS):
                pltpu.sync_copy(qbuf.at[s], pos_hbm.at[pl.ds(tb, _C), s])
            cc.wait()
            pltpu.sync_copy(cbuf, proxy_hbm.at[pl.ds(tb, _C)])
            return ()

        lax.fori_loop(0, _NCH, chunk, ())

    return _sc_gather


def kernel(img, aug_img, W1, b1, W2, b2, eW1, eb1, eW2, eb2, centroid, queue):
    xo = jnp.transpose(img, (0, 2, 3, 1)).reshape(-1, _FEAT)
    xa = jnp.transpose(aug_img, (0, 2, 3, 1)).reshape(-1, _FEAT)
    b1r = b1.reshape(1, _FEAT)
    b2r = b2.reshape(1, _HID)
    eb1r = eb1.reshape(1, _FEAT)
    eb2r = eb2.reshape(1, _HID)

    cn, c2 = pl.pallas_call(
        _prep_body,
        grid=(_K // _BK,),
        in_specs=[pl.BlockSpec((_BK, _HID), lambda i: (i, 0))],
        out_specs=[pl.BlockSpec((_BK, _HID), lambda i: (i, 0)),
                   pl.BlockSpec((_BK, 1), lambda i: (i, 0))],
        out_shape=[jax.ShapeDtypeStruct((_K, _HID), jnp.float32),
                   jax.ShapeDtypeStruct((_K, 1), jnp.float32)],
    )(centroid)

    z1, n1, z2, loss = pl.pallas_call(
        _head_body,
        grid=(_N // _BT_A,),
        in_specs=[pl.BlockSpec((_BT_A, _FEAT), lambda i: (i, 0)),
                  pl.BlockSpec((_BT_A, _FEAT), lambda i: (i, 0)),
                  pl.BlockSpec((_FEAT, _FEAT), lambda i: (0, 0)),
                  pl.BlockSpec((1, _FEAT), lambda i: (0, 0)),
                  pl.BlockSpec((_FEAT, _HID), lambda i: (0, 0)),
                  pl.BlockSpec((1, _HID), lambda i: (0, 0)),
                  pl.BlockSpec((_FEAT, _FEAT), lambda i: (0, 0)),
                  pl.BlockSpec((1, _FEAT), lambda i: (0, 0)),
                  pl.BlockSpec((_FEAT, _HID), lambda i: (0, 0)),
                  pl.BlockSpec((1, _HID), lambda i: (0, 0))],
        out_specs=[pl.BlockSpec((_BT_A, _HID), lambda i: (i, 0)),
                   pl.BlockSpec((_BT_A, _HID), lambda i: (i, 0)),
                   pl.BlockSpec((_BT_A, 1), lambda i: (i, 0)),
                   pl.BlockSpec((1, 1), lambda i: (0, 0))],
        out_shape=[jax.ShapeDtypeStruct((_N, _HID), jnp.float32),
                   jax.ShapeDtypeStruct((_N, _HID), jnp.float32),
                   jax.ShapeDtypeStruct((_N, 1), jnp.float32),
                   jax.ShapeDtypeStruct((1, 1), jnp.float32)],
    )(xo, xa, W1, b1r, W2, b2r, eW1, eb1r, eW2, eb2r)

    idx = pl.pallas_call(
        _vq_body,
        grid=(_N // _BT_B,),
        in_specs=[pl.BlockSpec((_BT_B, _HID), lambda i: (i, 0)),
                  pl.BlockSpec((_BT_B, 1), lambda i: (i, 0)),
                  pl.BlockSpec((_K, _HID), lambda i: (0, 0)),
                  pl.BlockSpec((1, _K), lambda i: (0, 0))],
        out_specs=pl.BlockSpec((_BT_B, 1), lambda i: (i, 0)),
        out_shape=jax.ShapeDtypeStruct((_N, 1), jnp.int32),
    )(n1, z2, cn, c2.reshape(1, _K))

    positives, pos_proxy = _make_sc_gather()(
        queue, centroid, idx.reshape(_N), z1)

    out = jnp.transpose(n1.reshape(8, 24, 24, _HID), (0, 3, 1, 2))
    loss1 = loss[0, 0]
    return (out, pos_proxy, positives, loss1)


# speedup vs baseline: 1.5562x; 1.1083x over previous
import functools

import jax
import jax.numpy as jnp
from jax import lax
from jax.experimental import pallas as pl
from jax.experimental.pallas import tpu as pltpu
from jax.experimental.pallas import tpu_sc as plsc

_FEAT = 768
_HID = 256
_K = 8192
_NS = 10
_MOM = 0.999
_N = 4608

_BT = 256
_BK = 1024

_NW = 32
_TW = _N // _NW
_C = 24
_NCH = _TW // _C


def _prep_body(cent_ref, w1_ref, b1_ref, w2_ref, b2_ref,
               ew1_ref, eb1_ref, ew2_ref, eb2_ref,
               cn_ref, c2_ref, uw1_ref, ub1_ref, uw2_ref, ub2_ref):
    c = cent_ref[...]
    nrm = jnp.sqrt(jnp.sum(c * c, axis=1, keepdims=True))
    cn = c / (nrm + 1e-12)
    cn_ref[...] = cn
    c2_ref[...] = jnp.sum(cn * cn, axis=1, keepdims=True)

    @pl.when(pl.program_id(0) == 0)
    def _():
        uw1_ref[...] = ew1_ref[...] * _MOM + w1_ref[...] * (1.0 - _MOM)
        ub1_ref[...] = eb1_ref[...] * _MOM + b1_ref[...] * (1.0 - _MOM)
        uw2_ref[...] = ew2_ref[...] * _MOM + w2_ref[...] * (1.0 - _MOM)
        ub2_ref[...] = eb2_ref[...] * _MOM + b2_ref[...] * (1.0 - _MOM)


def _fused_body(xo_ref, xa_ref, w1_ref, b1_ref, w2_ref, b2_ref,
                uw1_ref, ub1_ref, uw2_ref, ub2_ref, cn_ref, c2_ref,
                z1_ref, n1_ref, idx_ref, loss_ref):
    i = pl.program_id(0)
    xo = xo_ref[...]
    h1 = jnp.maximum(
        jnp.dot(xo, w1_ref[...], preferred_element_type=jnp.float32)
        + b1_ref[...], 0.0)
    z1 = jnp.dot(h1, w2_ref[...], preferred_element_type=jnp.float32) \
        + b2_ref[...]
    z1_ref[...] = z1
    nr1 = jnp.sqrt(jnp.sum(z1 * z1, axis=1, keepdims=True))
    n1 = z1 / (nr1 + 1e-12)
    n1_ref[...] = n1

    xa = xa_ref[...]
    h2 = jnp.maximum(
        jnp.dot(xa, uw1_ref[...], preferred_element_type=jnp.float32)
        + ub1_ref[...], 0.0)
    za = jnp.dot(h2, uw2_ref[...], preferred_element_type=jnp.float32) \
        + ub2_ref[...]
    nr2 = jnp.sqrt(jnp.sum(za * za, axis=1, keepdims=True))
    n2 = za / (nr2 + 1e-12)
    d = n1 - n2
    part = jnp.sum(d * d).reshape(1, 1)

    @pl.when(i == 0)
    def _():
        loss_ref[...] = jnp.zeros((1, 1), jnp.float32)
    loss_ref[...] += part
    @pl.when(i == pl.num_programs(0) - 1)
    def _():
        loss_ref[...] = loss_ref[...] / float(_N * _HID)

    mm = lax.dot_general(n1, cn_ref[...], (((1,), (1,)), ((), ())),
                         preferred_element_type=jnp.float32)
    dist = c2_ref[...] - 2.0 * mm
    dmin = jnp.min(dist, axis=1, keepdims=True)
    ii = lax.broadcasted_iota(jnp.int32, dist.shape, 1)
    cand = jnp.where(dist == dmin, ii, _K)
    idx_ref[...] = jnp.min(cand, axis=1, keepdims=True)


@functools.cache
def _make_sc_gather():
    @functools.partial(
        pl.kernel,
        out_type=[jax.ShapeDtypeStruct((_N, _NS, _HID), jnp.float32),
                  jax.ShapeDtypeStruct((_N, _HID), jnp.float32)],
        mesh=plsc.VectorSubcoreMesh(core_axis_name="c", subcore_axis_name="s"),
        scratch_types=[
            pltpu.VMEM((_C,), jnp.int32),
            pltpu.VMEM((_NS, _C, _HID), jnp.float32),
            pltpu.VMEM((_C, _HID), jnp.float32),
            pltpu.SemaphoreType.DMA,
            pltpu.SemaphoreType.DMA,
        ],
    )
    def _sc_gather(queue_hbm, cent_hbm, idx_hbm, z1_hbm, pos_hbm, proxy_hbm,
                   ic_v, qbuf, cbuf, semq, semc):
        wid = lax.axis_index("s") * 2 + lax.axis_index("c")
        base = wid * _TW

        def chunk(j, _):
            tb = base + j * _C
            pltpu.sync_copy(idx_hbm.at[pl.ds(tb, _C)], ic_v)
            cps = [pltpu.async_copy(queue_hbm.at[:, s].at[ic_v], qbuf.at[s],
                                    semq)
                   for s in range(_NS - 1)]
            cps.append(pltpu.async_copy(
                z1_hbm.at[pl.ds(tb, _C)], qbuf.at[_NS - 1], semq))
            cc = pltpu.async_copy(cent_hbm.at[ic_v], cbuf, semc)
            for w in cps:
                w.wait()
            for s in range(_NS):
                pltpu.sync_copy(qbuf.at[s], pos_hbm.at[pl.ds(tb, _C), s])
            cc.wait()
            pltpu.sync_copy(cbuf, proxy_hbm.at[pl.ds(tb, _C)])
            return ()

        lax.fori_loop(0, _NCH, chunk, ())

    return _sc_gather


def kernel(img, aug_img, W1, b1, W2, b2, eW1, eb1, eW2, eb2, centroid, queue):
    xo = jnp.transpose(img, (0, 2, 3, 1)).reshape(-1, _FEAT)
    xa = jnp.transpose(aug_img, (0, 2, 3, 1)).reshape(-1, _FEAT)
    b1r = b1.reshape(1, _FEAT)
    b2r = b2.reshape(1, _HID)
    eb1r = eb1.reshape(1, _FEAT)
    eb2r = eb2.reshape(1, _HID)

    _full = lambda shape: pl.BlockSpec(shape, lambda i: (0,) * len(shape))

    cn, c2, uW1, ub1, uW2, ub2 = pl.pallas_call(
        _prep_body,
        grid=(_K // _BK,),
        in_specs=[pl.BlockSpec((_BK, _HID), lambda i: (i, 0)),
                  _full((_FEAT, _FEAT)), _full((1, _FEAT)),
                  _full((_FEAT, _HID)), _full((1, _HID)),
                  _full((_FEAT, _FEAT)), _full((1, _FEAT)),
                  _full((_FEAT, _HID)), _full((1, _HID))],
        out_specs=[pl.BlockSpec((_BK, _HID), lambda i: (i, 0)),
                   pl.BlockSpec((_BK, 1), lambda i: (i, 0)),
                   _full((_FEAT, _FEAT)), _full((1, _FEAT)),
                   _full((_FEAT, _HID)), _full((1, _HID))],
        out_shape=[jax.ShapeDtypeStruct((_K, _HID), jnp.float32),
                   jax.ShapeDtypeStruct((_K, 1), jnp.float32),
                   jax.ShapeDtypeStruct((_FEAT, _FEAT), jnp.float32),
                   jax.ShapeDtypeStruct((1, _FEAT), jnp.float32),
                   jax.ShapeDtypeStruct((_FEAT, _HID), jnp.float32),
                   jax.ShapeDtypeStruct((1, _HID), jnp.float32)],
    )(centroid, W1, b1r, W2, b2r, eW1, eb1r, eW2, eb2r)

    z1, n1, idx, loss = pl.pallas_call(
        _fused_body,
        grid=(_N // _BT,),
        in_specs=[pl.BlockSpec((_BT, _FEAT), lambda i: (i, 0)),
                  pl.BlockSpec((_BT, _FEAT), lambda i: (i, 0)),
                  _full((_FEAT, _FEAT)), _full((1, _FEAT)),
                  _full((_FEAT, _HID)), _full((1, _HID)),
                  _full((_FEAT, _FEAT)), _full((1, _FEAT)),
                  _full((_FEAT, _HID)), _full((1, _HID)),
                  _full((_K, _HID)), _full((1, _K))],
        out_specs=[pl.BlockSpec((_BT, _HID), lambda i: (i, 0)),
                   pl.BlockSpec((_BT, _HID), lambda i: (i, 0)),
                   pl.BlockSpec((_BT, 1), lambda i: (i, 0)),
                   _full((1, 1))],
        out_shape=[jax.ShapeDtypeStruct((_N, _HID), jnp.float32),
                   jax.ShapeDtypeStruct((_N, _HID), jnp.float32),
                   jax.ShapeDtypeStruct((_N, 1), jnp.int32),
                   jax.ShapeDtypeStruct((1, 1), jnp.float32)],
    )(xo, xa, W1, b1r, W2, b2r, uW1, ub1, uW2, ub2, cn, c2.reshape(1, _K))

    positives, pos_proxy = _make_sc_gather()(
        queue, centroid, idx.reshape(_N), z1)

    out = jnp.transpose(n1.reshape(8, 24, 24, _HID), (0, 3, 1, 2))
    loss1 = loss[0, 0]
    return (out, pos_proxy, positives, loss1)


# speedup vs baseline: 1.5660x; 1.0063x over previous
import functools

import jax
import jax.numpy as jnp
from jax import lax
from jax.experimental import pallas as pl
from jax.experimental.pallas import tpu as pltpu
from jax.experimental.pallas import tpu_sc as plsc

_FEAT = 768
_HID = 256
_K = 8192
_NS = 10
_MOM = 0.999
_N = 4608

_BT = 256
_BK = 1024

_NW = 32
_TW = _N // _NW
_C = 24
_NCH = _TW // _C


def _prep_body(cent_ref, w1_ref, b1_ref, w2_ref, b2_ref,
               ew1_ref, eb1_ref, ew2_ref, eb2_ref,
               cn_ref, c2_ref, uw1_ref, ub1_ref, uw2_ref, ub2_ref):
    c = cent_ref[...]
    nrm = jnp.sqrt(jnp.sum(c * c, axis=1, keepdims=True))
    cn = c / (nrm + 1e-12)
    cn_ref[...] = cn
    c2_ref[...] = jnp.sum(cn * cn, axis=1, keepdims=True)

    @pl.when(pl.program_id(0) == 0)
    def _():
        uw1_ref[...] = ew1_ref[...] * _MOM + w1_ref[...] * (1.0 - _MOM)
        ub1_ref[...] = eb1_ref[...] * _MOM + b1_ref[...] * (1.0 - _MOM)
        uw2_ref[...] = ew2_ref[...] * _MOM + w2_ref[...] * (1.0 - _MOM)
        ub2_ref[...] = eb2_ref[...] * _MOM + b2_ref[...] * (1.0 - _MOM)


def _fused_body(xo_ref, xa_ref, w1_ref, b1_ref, w2_ref, b2_ref,
                uw1_ref, ub1_ref, uw2_ref, ub2_ref, cn_ref, c2_ref,
                z1_ref, n1_ref, idx_ref, loss_ref):
    i = pl.program_id(0)
    xo = xo_ref[...]
    h1 = jnp.maximum(
        jnp.dot(xo, w1_ref[...], preferred_element_type=jnp.float32)
        + b1_ref[...], 0.0)
    z1 = jnp.dot(h1, w2_ref[...], preferred_element_type=jnp.float32) \
        + b2_ref[...]
    z1_ref[...] = z1
    nr1 = jnp.sqrt(jnp.sum(z1 * z1, axis=1, keepdims=True))
    n1 = z1 / (nr1 + 1e-12)
    n1_ref[...] = n1

    xa = xa_ref[...]
    h2 = jnp.maximum(
        jnp.dot(xa, uw1_ref[...], preferred_element_type=jnp.float32)
        + ub1_ref[...], 0.0)
    za = jnp.dot(h2, uw2_ref[...], preferred_element_type=jnp.float32) \
        + ub2_ref[...]
    nr2 = jnp.sqrt(jnp.sum(za * za, axis=1, keepdims=True))
    n2 = za / (nr2 + 1e-12)
    d = n1 - n2
    part = jnp.sum(d * d).reshape(1, 1)

    @pl.when(i == 0)
    def _():
        loss_ref[...] = jnp.zeros((1, 1), jnp.float32)
    loss_ref[...] += part
    @pl.when(i == pl.num_programs(0) - 1)
    def _():
        loss_ref[...] = loss_ref[...] / float(_N * _HID)

    mm = lax.dot_general(n1, cn_ref[...], (((1,), (1,)), ((), ())),
                         preferred_element_type=jnp.float32)
    dist = c2_ref[...] - 2.0 * mm
    dmin = jnp.min(dist, axis=1, keepdims=True)
    ii = lax.broadcasted_iota(jnp.int32, dist.shape, 1)
    cand = jnp.where(dist == dmin, ii, _K)
    idx_ref[...] = jnp.min(cand, axis=1, keepdims=True)


@functools.cache
def _make_sc_gather():
    @functools.partial(
        pl.kernel,
        out_type=[jax.ShapeDtypeStruct((_N, _NS, _HID), jnp.float32),
                  jax.ShapeDtypeStruct((_N, _HID), jnp.float32)],
        mesh=plsc.VectorSubcoreMesh(core_axis_name="c", subcore_axis_name="s"),
        scratch_types=[
            pltpu.VMEM((_C,), jnp.int32),
            pltpu.VMEM((_C,), jnp.int32),
            pltpu.VMEM((_NS, _C, _HID), jnp.float32),
            pltpu.VMEM((_NS, _C, _HID), jnp.float32),
            pltpu.VMEM((_C, _HID), jnp.float32),
            pltpu.SemaphoreType.DMA,
            pltpu.SemaphoreType.DMA,
            pltpu.SemaphoreType.DMA,
            pltpu.SemaphoreType.DMA,
        ],
    )
    def _sc_gather(queue_hbm, cent_hbm, idx_hbm, z1_hbm, pos_hbm, proxy_hbm,
                   ic0, ic1, qbuf0, qbuf1, cbuf, semg0, semg1, semw0, semw1):
        wid = lax.axis_index("s") * 2 + lax.axis_index("c")
        base = wid * _TW
        ics = (ic0, ic1)
        qbufs = (qbuf0, qbuf1)
        semgs = (semg0, semg1)
        semws = (semw0, semw1)
        gath = [None, None]
        wout = [None, None]

        def start_chunk(j):
            b = j % 2
            if wout[b] is not None:
                for w in wout[b]:
                    w.wait()
                wout[b] = None
            tb = base + j * _C
            ic = ics[b]
            qb = qbufs[b]
            pltpu.sync_copy(idx_hbm.at[pl.ds(tb, _C)], ic)
            cps = [pltpu.async_copy(queue_hbm.at[:, s].at[ic], qb.at[s],
                                    semgs[b])
                   for s in range(_NS - 1)]
            cps.append(pltpu.async_copy(
                z1_hbm.at[pl.ds(tb, _C)], qb.at[_NS - 1], semgs[b]))
            gath[b] = cps

        def finish_chunk(j):
            b = j % 2
            tb = base + j * _C
            ic = ics[b]
            qb = qbufs[b]
            for w in gath[b]:
                w.wait()
            gath[b] = None
            wlist = [pltpu.async_copy(qb.at[s], pos_hbm.at[pl.ds(tb, _C), s],
                                      semws[b])
                     for s in range(_NS)]
            wout[b] = wlist
            pltpu.sync_copy(cent_hbm.at[ic], cbuf)
            pltpu.sync_copy(cbuf, proxy_hbm.at[pl.ds(tb, _C)])

        for j in range(_NCH):
            start_chunk(j)
            if j >= 1:
                finish_chunk(j - 1)
        finish_chunk(_NCH - 1)
        for wlist in wout:
            if wlist is not None:
                for w in wlist:
                    w.wait()

    return _sc_gather


def kernel(img, aug_img, W1, b1, W2, b2, eW1, eb1, eW2, eb2, centroid, queue):
    xo = jnp.transpose(img, (0, 2, 3, 1)).reshape(-1, _FEAT)
    xa = jnp.transpose(aug_img, (0, 2, 3, 1)).reshape(-1, _FEAT)
    b1r = b1.reshape(1, _FEAT)
    b2r = b2.reshape(1, _HID)
    eb1r = eb1.reshape(1, _FEAT)
    eb2r = eb2.reshape(1, _HID)

    _full = lambda shape: pl.BlockSpec(shape, lambda i: (0,) * len(shape))

    cn, c2, uW1, ub1, uW2, ub2 = pl.pallas_call(
        _prep_body,
        grid=(_K // _BK,),
        in_specs=[pl.BlockSpec((_BK, _HID), lambda i: (i, 0)),
                  _full((_FEAT, _FEAT)), _full((1, _FEAT)),
                  _full((_FEAT, _HID)), _full((1, _HID)),
                  _full((_FEAT, _FEAT)), _full((1, _FEAT)),
                  _full((_FEAT, _HID)), _full((1, _HID))],
        out_specs=[pl.BlockSpec((_BK, _HID), lambda i: (i, 0)),
                   pl.BlockSpec((_BK, 1), lambda i: (i, 0)),
                   _full((_FEAT, _FEAT)), _full((1, _FEAT)),
                   _full((_FEAT, _HID)), _full((1, _HID))],
        out_shape=[jax.ShapeDtypeStruct((_K, _HID), jnp.float32),
                   jax.ShapeDtypeStruct((_K, 1), jnp.float32),
                   jax.ShapeDtypeStruct((_FEAT, _FEAT), jnp.float32),
                   jax.ShapeDtypeStruct((1, _FEAT), jnp.float32),
                   jax.ShapeDtypeStruct((_FEAT, _HID), jnp.float32),
                   jax.ShapeDtypeStruct((1, _HID), jnp.float32)],
    )(centroid, W1, b1r, W2, b2r, eW1, eb1r, eW2, eb2r)

    z1, n1, idx, loss = pl.pallas_call(
        _fused_body,
        grid=(_N // _BT,),
        in_specs=[pl.BlockSpec((_BT, _FEAT), lambda i: (i, 0)),
                  pl.BlockSpec((_BT, _FEAT), lambda i: (i, 0)),
                  _full((_FEAT, _FEAT)), _full((1, _FEAT)),
                  _full((_FEAT, _HID)), _full((1, _HID)),
                  _full((_FEAT, _FEAT)), _full((1, _FEAT)),
                  _full((_FEAT, _HID)), _full((1, _HID)),
                  _full((_K, _HID)), _full((1, _K))],
        out_specs=[pl.BlockSpec((_BT, _HID), lambda i: (i, 0)),
                   pl.BlockSpec((_BT, _HID), lambda i: (i, 0)),
                   pl.BlockSpec((_BT, 1), lambda i: (i, 0)),
                   _full((1, 1))],
        out_shape=[jax.ShapeDtypeStruct((_N, _HID), jnp.float32),
                   jax.ShapeDtypeStruct((_N, _HID), jnp.float32),
                   jax.ShapeDtypeStruct((_N, 1), jnp.int32),
                   jax.ShapeDtypeStruct((1, 1), jnp.float32)],
    )(xo, xa, W1, b1r, W2, b2r, uW1, ub1, uW2, ub2, cn, c2.reshape(1, _K))

    positives, pos_proxy = _make_sc_gather()(
        queue, centroid, idx.reshape(_N), z1)

    out = jnp.transpose(n1.reshape(8, 24, 24, _HID), (0, 3, 1, 2))
    loss1 = loss[0, 0]
    return (out, pos_proxy, positives, loss1)


# speedup vs baseline: 1.5869x; 1.0133x over previous
import functools

import jax
import jax.numpy as jnp
from jax import lax
from jax.experimental import pallas as pl
from jax.experimental.pallas import tpu as pltpu
from jax.experimental.pallas import tpu_sc as plsc

_FEAT = 768
_HID = 256
_K = 8192
_NS = 10
_MOM = 0.999
_N = 4608

_BT = 256
_BK = 1024

_NW = 32
_TW = _N // _NW
_C = 24
_NCH = _TW // _C


def _prep_body(cent_ref, w1_ref, b1_ref, w2_ref, b2_ref,
               ew1_ref, eb1_ref, ew2_ref, eb2_ref,
               cn_ref, c2_ref, uw1_ref, ub1_ref, uw2_ref, ub2_ref):
    c = cent_ref[...]
    nrm = jnp.sqrt(jnp.sum(c * c, axis=1, keepdims=True))
    cn = c / (nrm + 1e-12)
    cn_ref[...] = cn
    c2_ref[...] = jnp.sum(cn * cn, axis=1, keepdims=True)

    @pl.when(pl.program_id(0) == 0)
    def _():
        uw1_ref[...] = ew1_ref[...] * _MOM + w1_ref[...] * (1.0 - _MOM)
        ub1_ref[...] = eb1_ref[...] * _MOM + b1_ref[...] * (1.0 - _MOM)
        uw2_ref[...] = ew2_ref[...] * _MOM + w2_ref[...] * (1.0 - _MOM)
        ub2_ref[...] = eb2_ref[...] * _MOM + b2_ref[...] * (1.0 - _MOM)


def _fused_body(xo_ref, xa_ref, w1_ref, b1_ref, w2_ref, b2_ref,
                uw1_ref, ub1_ref, uw2_ref, ub2_ref, cn_ref, c2_ref,
                z1_ref, n1_ref, idx_ref, loss_ref):
    i = pl.program_id(0)
    xo = xo_ref[...]
    h1 = jnp.maximum(
        jnp.dot(xo, w1_ref[...], preferred_element_type=jnp.float32)
        + b1_ref[...], 0.0)
    z1 = jnp.dot(h1, w2_ref[...], preferred_element_type=jnp.float32) \
        + b2_ref[...]
    z1_ref[...] = z1
    nr1 = jnp.sqrt(jnp.sum(z1 * z1, axis=1, keepdims=True))
    n1 = z1 / (nr1 + 1e-12)
    n1_ref[...] = n1

    xa = xa_ref[...]
    h2 = jnp.maximum(
        jnp.dot(xa, uw1_ref[...], preferred_element_type=jnp.float32)
        + ub1_ref[...], 0.0)
    za = jnp.dot(h2, uw2_ref[...], preferred_element_type=jnp.float32) \
        + ub2_ref[...]
    nr2 = jnp.sqrt(jnp.sum(za * za, axis=1, keepdims=True))
    n2 = za / (nr2 + 1e-12)
    d = n1 - n2
    part = jnp.sum(d * d).reshape(1, 1)

    @pl.when(i == 0)
    def _():
        loss_ref[...] = jnp.zeros((1, 1), jnp.float32)
    loss_ref[...] += part
    @pl.when(i == pl.num_programs(0) - 1)
    def _():
        loss_ref[...] = loss_ref[...] / float(_N * _HID)

    mm = lax.dot_general(n1, cn_ref[...], (((1,), (1,)), ((), ())),
                         preferred_element_type=jnp.float32)
    dist = c2_ref[...] - 2.0 * mm
    dmin = jnp.min(dist, axis=1, keepdims=True)
    ii = lax.broadcasted_iota(jnp.int32, dist.shape, 1)
    cand = jnp.where(dist == dmin, ii, _K)
    idx_ref[...] = jnp.min(cand, axis=1, keepdims=True)


@functools.cache
def _make_sc_gather():
    @functools.partial(
        pl.kernel,
        out_type=[jax.ShapeDtypeStruct((_N, _NS, _HID), jnp.float32),
                  jax.ShapeDtypeStruct((_N, _HID), jnp.float32)],
        mesh=plsc.VectorSubcoreMesh(core_axis_name="c", subcore_axis_name="s"),
        scratch_types=[
            pltpu.VMEM((_C,), jnp.int32),
            pltpu.VMEM((_C,), jnp.int32),
            pltpu.VMEM((_C, 8, _HID), jnp.float32),
            pltpu.VMEM((_C, 8, _HID), jnp.float32),
            pltpu.VMEM((2, _C, _HID), jnp.float32),
            pltpu.VMEM((2, _C, _HID), jnp.float32),
            pltpu.VMEM((_C, _HID), jnp.float32),
            pltpu.SemaphoreType.DMA,
            pltpu.SemaphoreType.DMA,
            pltpu.SemaphoreType.DMA,
            pltpu.SemaphoreType.DMA,
        ],
    )
    def _sc_gather(queue_hbm, cent_hbm, idx_hbm, z1_hbm, pos_hbm, proxy_hbm,
                   ic0, ic1, qs0, qs1, qt0, qt1, cbuf,
                   semg0, semg1, semw0, semw1):
        wid = lax.axis_index("s") * 2 + lax.axis_index("c")
        base = wid * _TW
        ics = (ic0, ic1)
        qss = (qs0, qs1)
        qts = (qt0, qt1)
        semgs = (semg0, semg1)
        semws = (semw0, semw1)
        gath = [None, None]
        wout = [None, None]

        def start_chunk(j):
            b = j % 2
            if wout[b] is not None:
                for w in wout[b]:
                    w.wait()
                wout[b] = None
            tb = base + j * _C
            ic = ics[b]
            pltpu.sync_copy(idx_hbm.at[pl.ds(tb, _C)], ic)
            gath[b] = [
                pltpu.async_copy(queue_hbm.at[:, pl.ds(0, 8)].at[ic],
                                 qss[b], semgs[b]),
                pltpu.async_copy(queue_hbm.at[:, _NS - 2].at[ic],
                                 qts[b].at[0], semgs[b]),
                pltpu.async_copy(z1_hbm.at[pl.ds(tb, _C)],
                                 qts[b].at[1], semgs[b]),
            ]

        def finish_chunk(j):
            b = j % 2
            tb = base + j * _C
            ic = ics[b]
            for w in gath[b]:
                w.wait()
            gath[b] = None
            wout[b] = [
                pltpu.async_copy(qss[b], pos_hbm.at[pl.ds(tb, _C),
                                                    pl.ds(0, 8)], semws[b]),
                pltpu.async_copy(qts[b].at[0],
                                 pos_hbm.at[pl.ds(tb, _C), _NS - 2], semws[b]),
                pltpu.async_copy(qts[b].at[1],
                                 pos_hbm.at[pl.ds(tb, _C), _NS - 1], semws[b]),
            ]
            pltpu.sync_copy(cent_hbm.at[ic], cbuf)
            pltpu.sync_copy(cbuf, proxy_hbm.at[pl.ds(tb, _C)])

        for j in range(_NCH):
            start_chunk(j)
            if j >= 1:
                finish_chunk(j - 1)
        finish_chunk(_NCH - 1)
        for wlist in wout:
            if wlist is not None:
                for w in wlist:
                    w.wait()

    return _sc_gather


def kernel(img, aug_img, W1, b1, W2, b2, eW1, eb1, eW2, eb2, centroid, queue):
    xo = jnp.transpose(img, (0, 2, 3, 1)).reshape(-1, _FEAT)
    xa = jnp.transpose(aug_img, (0, 2, 3, 1)).reshape(-1, _FEAT)
    b1r = b1.reshape(1, _FEAT)
    b2r = b2.reshape(1, _HID)
    eb1r = eb1.reshape(1, _FEAT)
    eb2r = eb2.reshape(1, _HID)

    _full = lambda shape: pl.BlockSpec(shape, lambda i: (0,) * len(shape))

    cn, c2, uW1, ub1, uW2, ub2 = pl.pallas_call(
        _prep_body,
        grid=(_K // _BK,),
        in_specs=[pl.BlockSpec((_BK, _HID), lambda i: (i, 0)),
                  _full((_FEAT, _FEAT)), _full((1, _FEAT)),
                  _full((_FEAT, _HID)), _full((1, _HID)),
                  _full((_FEAT, _FEAT)), _full((1, _FEAT)),
                  _full((_FEAT, _HID)), _full((1, _HID))],
        out_specs=[pl.BlockSpec((_BK, _HID), lambda i: (i, 0)),
                   pl.BlockSpec((_BK, 1), lambda i: (i, 0)),
                   _full((_FEAT, _FEAT)), _full((1, _FEAT)),
                   _full((_FEAT, _HID)), _full((1, _HID))],
        out_shape=[jax.ShapeDtypeStruct((_K, _HID), jnp.float32),
                   jax.ShapeDtypeStruct((_K, 1), jnp.float32),
                   jax.ShapeDtypeStruct((_FEAT, _FEAT), jnp.float32),
                   jax.ShapeDtypeStruct((1, _FEAT), jnp.float32),
                   jax.ShapeDtypeStruct((_FEAT, _HID), jnp.float32),
                   jax.ShapeDtypeStruct((1, _HID), jnp.float32)],
    )(centroid, W1, b1r, W2, b2r, eW1, eb1r, eW2, eb2r)

    z1, n1, idx, loss = pl.pallas_call(
        _fused_body,
        grid=(_N // _BT,),
        in_specs=[pl.BlockSpec((_BT, _FEAT), lambda i: (i, 0)),
                  pl.BlockSpec((_BT, _FEAT), lambda i: (i, 0)),
                  _full((_FEAT, _FEAT)), _full((1, _FEAT)),
                  _full((_FEAT, _HID)), _full((1, _HID)),
                  _full((_FEAT, _FEAT)), _full((1, _FEAT)),
                  _full((_FEAT, _HID)), _full((1, _HID)),
                  _full((_K, _HID)), _full((1, _K))],
        out_specs=[pl.BlockSpec((_BT, _HID), lambda i: (i, 0)),
                   pl.BlockSpec((_BT, _HID), lambda i: (i, 0)),
                   pl.BlockSpec((_BT, 1), lambda i: (i, 0)),
                   _full((1, 1))],
        out_shape=[jax.ShapeDtypeStruct((_N, _HID), jnp.float32),
                   jax.ShapeDtypeStruct((_N, _HID), jnp.float32),
                   jax.ShapeDtypeStruct((_N, 1), jnp.int32),
                   jax.ShapeDtypeStruct((1, 1), jnp.float32)],
    )(xo, xa, W1, b1r, W2, b2r, uW1, ub1, uW2, ub2, cn, c2.reshape(1, _K))

    positives, pos_proxy = _make_sc_gather()(
        queue, centroid, idx.reshape(_N), z1)

    out = jnp.transpose(n1.reshape(8, 24, 24, _HID), (0, 3, 1, 2))
    loss1 = loss[0, 0]
    return (out, pos_proxy, positives, loss1)
